# Initial kernel scaffold; baseline (speedup 1.0000x reference)
import jax
import jax.numpy as jnp
from jax.experimental import pallas as pl


def kernel(oldx, W_in, b_in, W_out):
    raise NotImplementedError("write your pallas kernel here")


# speedup vs baseline: 3.3175x; 3.3175x over previous
import numpy as np
import jax
import jax.numpy as jnp
from jax import lax
from jax.experimental import pallas as pl
from jax.experimental.pallas import tpu as pltpu

DIM = 2048
DEPTH = 7
PAR = 16
NN = 255
NP = 256
B_TOT = 8192


def _build_H() -> np.ndarray:
    H = np.zeros((NP, NP), dtype=np.float32)
    for n in range(NN):
        m = n
        while m > 0:
            parent = (m - 1) // 2
            c = (m - 1) % 2
            H[parent, n] += 2 * c - 1
            H[NN, n] -= c
            m = parent
    H[NN, NN] = 1.0
    return H


_H = _build_H()

_BT1 = 1024
_BT2 = 1024


def _k1_body(x_ref, w_ref, b_ref, h_ref, acts_ref):
    x = x_ref[...]
    w = w_ref[0]
    logits = lax.dot_general(
        x, w, (((1,), (1,)), ((), ())),
        preferred_element_type=jnp.float32,
    ) + b_ref[0]
    dec = (logits > 0).astype(jnp.float32)
    S = jnp.dot(dec, h_ref[...], preferred_element_type=jnp.float32)
    silu = logits * jax.nn.sigmoid(logits)
    acts_ref[0] = jnp.where(S == 0.0, silu, 0.0)


def _k2_body(acts_ref, w_ref, out_ref):
    p = pl.program_id(1)

    @pl.when(p == 0)
    def _():
        out_ref[...] = jnp.zeros_like(out_ref)

    out_ref[...] += jnp.dot(
        acts_ref[0], w_ref[0],
        preferred_element_type=jnp.float32,
    )


def kernel(oldx, W_in, b_in, W_out):
    x = oldx.reshape(-1, DIM)
    B = x.shape[0]

    Wr = jnp.pad(W_in.reshape(PAR, NN, DIM), ((0, 0), (0, 1), (0, 0)))
    br = jnp.pad(b_in.reshape(PAR, 1, NN), ((0, 0), (0, 0), (0, 1)),
                 constant_values=1.0)
    Wo = jnp.pad(W_out.T.reshape(PAR, NN, DIM), ((0, 0), (0, 1), (0, 0)))
    H = jnp.asarray(_H)

    bt1 = min(_BT1, B)
    nbt1 = B // bt1
    acts = pl.pallas_call(
        _k1_body,
        grid=(nbt1, PAR),
        in_specs=[
            pl.BlockSpec((bt1, DIM), lambda i, p: (i, 0)),
            pl.BlockSpec((1, NP, DIM), lambda i, p: (p, 0, 0)),
            pl.BlockSpec((1, 1, NP), lambda i, p: (p, 0, 0)),
            pl.BlockSpec((NP, NP), lambda i, p: (0, 0)),
        ],
        out_specs=pl.BlockSpec((1, bt1, NP), lambda i, p: (p, i, 0)),
        out_shape=jax.ShapeDtypeStruct((PAR, B, NP), jnp.float32),
    )(x, Wr, br, H)

    bt2 = min(_BT2, B)
    nbt2 = B // bt2
    out = pl.pallas_call(
        _k2_body,
        grid=(nbt2, PAR),
        in_specs=[
            pl.BlockSpec((1, bt2, NP), lambda i, p: (p, i, 0)),
            pl.BlockSpec((1, NP, DIM), lambda i, p: (p, 0, 0)),
        ],
        out_specs=pl.BlockSpec((bt2, DIM), lambda i, p: (i, 0)),
        out_shape=jax.ShapeDtypeStruct((B, DIM), jnp.float32),
        compiler_params=pltpu.CompilerParams(
            dimension_semantics=("parallel", "arbitrary")
        ),
    )(acts, Wo)

    return out.reshape(oldx.shape)
